# baseline (speedup 1.0000x reference)
import functools

import jax
import jax.numpy as jnp
from jax import lax
from jax.experimental import pallas as pl
from jax.experimental.pallas import tpu as pltpu
from jax.experimental.pallas import tpu_sc as plsc

NUM_CLASSES = 100000
IMAGE_SIZE = 16
NUM_CHANNELS = 3
BATCH = 16384
EMB_DIM = NUM_CHANNELS * IMAGE_SIZE * IMAGE_SIZE
CHAN = IMAGE_SIZE * IMAGE_SIZE

_NC = 2
_NS = 16
_NW = _NC * _NS
_ROWS_PER_W = BATCH // _NW
_CHUNK = 64
_NCHUNK = _ROWS_PER_W // _CHUNK

_BR = 512
_NBLK = BATCH // _BR


def _sc_gather(label, table):
    mesh = plsc.VectorSubcoreMesh(core_axis_name="c", subcore_axis_name="s")

    @functools.partial(
        pl.kernel,
        mesh=mesh,
        out_type=jax.ShapeDtypeStruct((BATCH, EMB_DIM), jnp.float32),
        scratch_types=[
            pltpu.VMEM((_CHUNK,), jnp.int32),
            pltpu.VMEM((_CHUNK, EMB_DIM), jnp.float32),
            pltpu.SemaphoreType.DMA,
        ],
    )
    def gather_kernel(label_hbm, table_hbm, out_hbm, idx_v, rows_v, sem):
        wid = lax.axis_index("s") * _NC + lax.axis_index("c")
        base = wid * _ROWS_PER_W
        for c in range(_NCHUNK):
            off = base + c * _CHUNK
            pltpu.sync_copy(label_hbm.at[pl.ds(off, _CHUNK)], idx_v)
            pltpu.async_copy(table_hbm.at[idx_v], rows_v, sem).wait()
            pltpu.sync_copy(rows_v, out_hbm.at[pl.ds(off, _CHUNK)])

    return gather_kernel(label, table)


def _stats_body(x_ref, w_ref, b_ref, params_ref, acc_ref):
    i = pl.program_id(0)

    @pl.when(i == 0)
    def _init():
        acc_ref[...] = jnp.zeros_like(acc_ref)

    x = x_ref[...]
    acc_ref[0:1, :] += jnp.sum(x, axis=0, keepdims=True)
    acc_ref[1:2, :] += jnp.sum(x * x, axis=0, keepdims=True)

    @pl.when(i == pl.num_programs(0) - 1)
    def _finish():
        n = jnp.float32(BATCH * CHAN)
        for c in range(NUM_CHANNELS):
            lo = c * CHAN
            s_c = jnp.sum(acc_ref[0:1, lo : lo + CHAN])
            ss_c = jnp.sum(acc_ref[1:2, lo : lo + CHAN])
            mean = s_c / n
            var = ss_c / n - mean * mean
            scale = lax.rsqrt(var + 1e-5) * w_ref[c]
            shift = b_ref[c] - mean * scale
            params_ref[0:1, lo : lo + CHAN] = jnp.full((1, CHAN), scale, jnp.float32)
            params_ref[1:2, lo : lo + CHAN] = jnp.full((1, CHAN), shift, jnp.float32)


def _norm_body(x_ref, params_ref, o_ref):
    o_ref[...] = x_ref[...] * params_ref[0:1, :] + params_ref[1:2, :]


def kernel(label, table, bn_weight, bn_bias):
    gathered = _sc_gather(label, table)

    params = pl.pallas_call(
        _stats_body,
        grid=(_NBLK,),
        in_specs=[
            pl.BlockSpec((_BR, EMB_DIM), lambda i: (i, 0)),
            pl.BlockSpec(memory_space=pltpu.SMEM),
            pl.BlockSpec(memory_space=pltpu.SMEM),
        ],
        out_specs=pl.BlockSpec((2, EMB_DIM), lambda i: (0, 0)),
        out_shape=jax.ShapeDtypeStruct((2, EMB_DIM), jnp.float32),
        scratch_shapes=[pltpu.VMEM((2, EMB_DIM), jnp.float32)],
    )(gathered, bn_weight, bn_bias)

    out = pl.pallas_call(
        _norm_body,
        grid=(_NBLK,),
        in_specs=[
            pl.BlockSpec((_BR, EMB_DIM), lambda i: (i, 0)),
            pl.BlockSpec((2, EMB_DIM), lambda i: (0, 0)),
        ],
        out_specs=pl.BlockSpec((_BR, EMB_DIM), lambda i: (i, 0)),
        out_shape=jax.ShapeDtypeStruct((BATCH, EMB_DIM), jnp.float32),
    )(gathered, params)

    return out.reshape(-1, NUM_CHANNELS, IMAGE_SIZE, IMAGE_SIZE)
